# baseline (speedup 1.0000x reference)
import jax
import jax.numpy as jnp
from jax.experimental import pallas as pl

B = 512
P = 196
H = 768
M = 512
BB = 32
INV_P = 1.0 / P


def _mean_fifo_body(states_ref, ts_ref, mem_ref, ts_out_ref):
    i = pl.program_id(0)
    mem_ref[:] = jnp.sum(states_ref[:], axis=1) * INV_P

    @pl.when(i == 0)
    def _():
        ts_out_ref[:] = ts_ref[:]


def kernel(states, timestamp, memory_states, memory_timestamps):
    ts2 = timestamp.reshape(1, B)
    new_mem, new_ts = pl.pallas_call(
        _mean_fifo_body,
        grid=(B // BB,),
        in_specs=[
            pl.BlockSpec((BB, P, H), lambda i: (i, 0, 0)),
            pl.BlockSpec((1, B), lambda i: (0, 0)),
        ],
        out_specs=[
            pl.BlockSpec((BB, H), lambda i: (i, 0)),
            pl.BlockSpec((1, B), lambda i: (0, 0)),
        ],
        out_shape=[
            jax.ShapeDtypeStruct((M, H), jnp.float32),
            jax.ShapeDtypeStruct((1, B), jnp.int32),
        ],
    )(states, ts2)
    new_ts = new_ts.reshape(B).astype(memory_timestamps.dtype)
    new_valid = jnp.ones((M,), dtype=jnp.bool_)
    new_ptr = jnp.full((1,), B % M, dtype=jnp.int32)
    new_count = jnp.full((1,), min(B, M), dtype=jnp.int32)
    return (new_mem, new_ts, new_valid, new_ptr, new_count)
